# baseline (speedup 1.0000x reference)
import jax
import jax.numpy as jnp
from jax.experimental import pallas as pl
from jax.experimental.pallas import tpu as pltpu

_FILL = -10000.0
_VOCAB = 100000
_T = 2048
_H = 1024
_RESTRICTED = 65
_ROWS = 32
_SLOTS = 2


def _body(hs_ref, wt_ref, out_ref, scratch, logits, sem):
    i = pl.program_id(0)
    n = pl.num_programs(0)
    slot = jax.lax.rem(i, _SLOTS)

    @pl.when(i == 0)
    def _matmul():
        logits[...] = jnp.dot(hs_ref[...], wt_ref[...],
                              preferred_element_type=jnp.float32)

    @pl.when(i >= _SLOTS)
    def _wait_prev():
        pltpu.make_async_copy(
            scratch.at[slot],
            out_ref.at[0, pl.ds((i - _SLOTS) * _ROWS, _ROWS), :],
            sem.at[slot],
        ).wait()

    @pl.when(i < _SLOTS)
    def _fill():
        scratch[slot] = jnp.full((_ROWS, _VOCAB), _FILL, dtype=jnp.float32)

    scratch[slot, :, 100:164] = logits[pl.ds(i * _ROWS, _ROWS), 0:64]
    scratch[slot, :, 999:1000] = logits[pl.ds(i * _ROWS, _ROWS), 64:65]

    pltpu.make_async_copy(
        scratch.at[slot],
        out_ref.at[0, pl.ds(i * _ROWS, _ROWS), :],
        sem.at[slot],
    ).start()

    @pl.when(i == n - 1)
    def _drain():
        pltpu.make_async_copy(
            scratch.at[1 - slot],
            out_ref.at[0, pl.ds((i - 1) * _ROWS, _ROWS), :],
            sem.at[1 - slot],
        ).wait()
        pltpu.make_async_copy(
            scratch.at[slot],
            out_ref.at[0, pl.ds(i * _ROWS, _ROWS), :],
            sem.at[slot],
        ).wait()


def kernel(hidden_states, W):
    B, T, H = hidden_states.shape
    hs = hidden_states.reshape(T, H).astype(jnp.float32)
    wt = jnp.zeros((H, 128), dtype=jnp.float32).at[:, :_RESTRICTED].set(
        W.astype(jnp.float32).T)

    n_steps = T // _ROWS
    out = pl.pallas_call(
        _body,
        grid=(n_steps,),
        in_specs=[
            pl.BlockSpec((T, H), lambda i: (0, 0)),
            pl.BlockSpec((H, 128), lambda i: (0, 0)),
        ],
        out_specs=pl.BlockSpec(memory_space=pltpu.MemorySpace.HBM),
        out_shape=jax.ShapeDtypeStruct((B, T, _VOCAB), jnp.float32),
        scratch_shapes=[
            pltpu.VMEM((_SLOTS, _ROWS, _VOCAB), jnp.float32),
            pltpu.VMEM((_T, 128), jnp.float32),
            pltpu.SemaphoreType.DMA((_SLOTS,)),
        ],
    )(hs, wt)
    return out


# speedup vs baseline: 1.0038x; 1.0038x over previous
import jax
import jax.numpy as jnp
from jax.experimental import pallas as pl

_FILL = -10000.0
_VOCAB = 100000
_RESTRICTED = 65
_VB = 2048


def _body(hs_ref, wt_ref, out_ref):
    j = pl.program_id(0)
    out_ref[...] = jnp.full(out_ref.shape, _FILL, dtype=jnp.float32)

    @pl.when(j == 0)
    def _scatter():
        logits = jnp.dot(hs_ref[0], wt_ref[...],
                         preferred_element_type=jnp.float32)
        out_ref[0, :, 100:164] = logits[:, 0:64]
        out_ref[0, :, 999:1000] = logits[:, 64:65]


def kernel(hidden_states, W):
    B, T, H = hidden_states.shape
    hs = hidden_states.astype(jnp.float32)
    wt = jnp.zeros((H, 128), dtype=jnp.float32).at[:, :_RESTRICTED].set(
        W.astype(jnp.float32).T)

    n_blocks = pl.cdiv(_VOCAB, _VB)
    out = pl.pallas_call(
        _body,
        grid=(n_blocks,),
        in_specs=[
            pl.BlockSpec((1, T, H), lambda j: (0, 0, 0)),
            pl.BlockSpec((H, 128), lambda j: (0, 0)),
        ],
        out_specs=pl.BlockSpec((1, T, _VB), lambda j: (0, 0, j)),
        out_shape=jax.ShapeDtypeStruct((B, T, _VOCAB), jnp.float32),
    )(hs, wt)
    return out


# speedup vs baseline: 3.8255x; 3.8110x over previous
import jax
import jax.numpy as jnp
from jax.experimental import pallas as pl

_FILL = -10000.0
_VOCAB = 100000
_RESTRICTED = 65
_VB = 2048


def _body(hs_ref, w_ref, out_ref):
    j = pl.program_id(0)
    out_ref[...] = jnp.full(out_ref.shape, _FILL, dtype=jnp.float32)

    @pl.when(j == 0)
    def _scatter():
        logits_t = jax.lax.dot_general(
            w_ref[...], hs_ref[0],
            dimension_numbers=(((1,), (1,)), ((), ())),
            preferred_element_type=jnp.float32)
        out_ref[100:164, :] = logits_t[0:64, :]
        out_ref[999:1000, :] = logits_t[64:65, :]


def kernel(hidden_states, W):
    B, T, H = hidden_states.shape
    hs = hidden_states.astype(jnp.float32)
    w_pad = jnp.zeros((128, H), dtype=jnp.float32).at[:_RESTRICTED].set(
        W.astype(jnp.float32))

    n_blocks = pl.cdiv(_VOCAB, _VB)
    out_t = pl.pallas_call(
        _body,
        grid=(n_blocks,),
        in_specs=[
            pl.BlockSpec((1, T, H), lambda j: (0, 0, 0)),
            pl.BlockSpec((128, H), lambda j: (0, 0)),
        ],
        out_specs=pl.BlockSpec((_VB, T), lambda j: (j, 0)),
        out_shape=jax.ShapeDtypeStruct((_VOCAB, T), jnp.float32),
    )(hs, w_pad)
    return jnp.swapaxes(out_t, 0, 1)[None]
